# baseline (speedup 1.0000x reference)
import functools

import jax
import jax.numpy as jnp
from jax.experimental import pallas as pl
from jax.experimental.pallas import tpu as pltpu


def _body(bits_ref, pat_ref, val_ref, out_ref, lhs_ref):
    nb, m, p = pat_ref.shape
    b = bits_ref.shape[0]
    lhs_ref[b:, :] = jnp.ones((8, p), jnp.bfloat16)
    iota_row = jax.lax.broadcasted_iota(
        jnp.int32, (1, m), 1).astype(jnp.float32)
    for j in range(nb):
        lhs_ref[:b, :] = bits_ref[:, j, :].astype(jnp.bfloat16)
        one = jnp.bfloat16(1)
        zero = jnp.bfloat16(0)
        a_bf = jnp.where(pat_ref[j].astype(jnp.bfloat16) > 0, one, zero)
        cross = jax.lax.dot_general(
            lhs_ref[...], a_bf,
            dimension_numbers=(((1,), (1,)), ((), ())),
            preferred_element_type=jnp.float32)
        key = (cross[b:b + 1, :] * float(m) + iota_row) \
            - (2.0 * m) * cross[:b, :]
        min_key = jnp.min(key, axis=1, keepdims=True)
        onehot = jnp.where(key == min_key, one, zero)
        v_hard = jnp.where(val_ref[j].astype(jnp.bfloat16) > 0, one, zero)
        out_ref[j] = jax.lax.dot_general(
            v_hard, onehot,
            dimension_numbers=(((0,), (1,)), ((), ())),
            preferred_element_type=jnp.float32)


@functools.partial(jax.jit, static_argnames=("block_n",))
def _vgram_lookup(bits, pattern_logits, value_logits, block_n=32):
    b, n, p = bits.shape
    _, m, d = value_logits.shape
    grid = (n // block_n,)
    out_t = pl.pallas_call(
        _body,
        grid=grid,
        in_specs=[
            pl.BlockSpec((b, block_n, p), lambda i: (0, i, 0)),
            pl.BlockSpec((block_n, m, p), lambda i: (i, 0, 0)),
            pl.BlockSpec((block_n, m, d), lambda i: (i, 0, 0)),
        ],
        out_specs=pl.BlockSpec((block_n, d, b), lambda i: (i, 0, 0)),
        out_shape=jax.ShapeDtypeStruct((n, d, b), jnp.float32),
        scratch_shapes=[pltpu.VMEM((b + 8, p), jnp.bfloat16)],
    )(bits, pattern_logits, value_logits)
    return out_t.transpose(2, 0, 1)


def kernel(bits, pattern_logits, value_logits):
    return _vgram_lookup(bits, pattern_logits, value_logits)


# speedup vs baseline: 1.3332x; 1.3332x over previous
import functools

import jax
import jax.numpy as jnp
from jax.experimental import pallas as pl
from jax.experimental.pallas import tpu as pltpu


def _body(bits_ref, pat_ref, val_ref, out_ref, lhs_ref):
    nb, m, p = pat_ref.shape
    b = bits_ref.shape[0]
    lhs_ref[b:, :] = jnp.ones((8, p), jnp.bfloat16)
    iota_row = jax.lax.broadcasted_iota(
        jnp.int32, (1, m), 1).astype(jnp.float32)
    for j in range(nb):
        lhs_ref[:b, :] = bits_ref[:, j, :].astype(jnp.bfloat16)
        one = jnp.bfloat16(1)
        zero = jnp.bfloat16(0)
        a_bf = jnp.where(pat_ref[j].astype(jnp.bfloat16) > 0, one, zero)
        cross = jax.lax.dot_general(
            lhs_ref[...], a_bf,
            dimension_numbers=(((1,), (1,)), ((), ())),
            preferred_element_type=jnp.float32)
        key = (cross[b:b + 1, :] * float(m) + iota_row) \
            - (2.0 * m) * cross[:b, :]
        min_key = jnp.min(key, axis=1, keepdims=True)
        onehot = jnp.where(key == min_key, jnp.float32(1),
                           jnp.float32(0)).astype(jnp.bfloat16)
        v_hard = jnp.where(val_ref[j].astype(jnp.bfloat16) > 0, one, zero)
        out_ref[j] = jax.lax.dot_general(
            v_hard, onehot,
            dimension_numbers=(((0,), (1,)), ((), ())),
            preferred_element_type=jnp.float32)


@functools.partial(jax.jit, static_argnames=("block_n",))
def _vgram_lookup(bits, pattern_logits, value_logits, block_n=32):
    b, n, p = bits.shape
    _, m, d = value_logits.shape
    grid = (n // block_n,)
    out_t = pl.pallas_call(
        _body,
        grid=grid,
        in_specs=[
            pl.BlockSpec((b, block_n, p), lambda i: (0, i, 0)),
            pl.BlockSpec((block_n, m, p), lambda i: (i, 0, 0)),
            pl.BlockSpec((block_n, m, d), lambda i: (i, 0, 0)),
        ],
        out_specs=pl.BlockSpec((block_n, d, b), lambda i: (i, 0, 0)),
        out_shape=jax.ShapeDtypeStruct((n, d, b), jnp.float32),
        scratch_shapes=[pltpu.VMEM((b + 8, p), jnp.bfloat16)],
    )(bits, pattern_logits, value_logits)
    return out_t.transpose(2, 0, 1)


def kernel(bits, pattern_logits, value_logits):
    return _vgram_lookup(bits, pattern_logits, value_logits)
